# baseline (speedup 1.0000x reference)
import functools

import jax
import jax.numpy as jnp
from jax import lax
from jax.experimental import pallas as pl
from jax.experimental.pallas import tpu as pltpu
from jax.experimental.pallas import tpu_sc as plsc

VOCAB = 1000000
EMB = 64
B = 16384
NEG = 20

NC = 2
NS = 16
NW = NC * NS
BPW = B // NW
GR = 128
NCH = BPW * NEG // GR


def _sc_rows(in_hbm, out_hbm, tgt_hbm, ctx_hbm, t_out, c_out,
             tidx_v, cidx_v, tsem, csem):
    wid = lax.axis_index("s") * NC + lax.axis_index("c")
    base = wid * BPW
    pltpu.sync_copy(tgt_hbm.at[pl.ds(base, BPW)], tidx_v)
    pltpu.sync_copy(ctx_hbm.at[pl.ds(base, BPW)], cidx_v)

    def fire(g, carry):
        i0 = g * 16
        tv = tidx_v[pl.ds(i0, 16)]
        cv = cidx_v[pl.ds(i0, 16)]
        for k in range(16):
            v = tv[k]
            pltpu.async_copy(in_hbm.at[v >> 3, v & 7], t_out.at[base + i0 + k],
                             tsem)
            w = cv[k]
            pltpu.async_copy(out_hbm.at[w >> 3, w & 7],
                             c_out.at[base + i0 + k], csem)
        return carry

    lax.fori_loop(0, BPW // 16, fire, 0)

    def drain(i, carry):
        pltpu.make_async_copy(in_hbm.at[0, 0], t_out.at[base], tsem).wait()
        pltpu.make_async_copy(out_hbm.at[0, 0], c_out.at[base], csem).wait()
        return carry

    lax.fori_loop(0, BPW, drain, 0)


def _sc_neg(out_hbm, neg_hbm, scat_hbm, zer_hbm, n_out,
            acc_sh, nidx_v, sidx_v, nbuf0_v, nbuf1_v, sem0, sem1, ssem):
    sid = lax.axis_index("s")
    wid = sid * NC + lax.axis_index("c")
    base = wid * BPW

    pltpu.sync_copy(zer_hbm, acc_sh.at[pl.ds(sid * BPW, BPW)])
    pltpu.sync_copy(neg_hbm.at[pl.ds(wid * NCH, NCH)], nidx_v)
    pltpu.sync_copy(scat_hbm.at[pl.ds(wid * NCH, NCH)], sidx_v)
    plsc.subcore_barrier()

    pltpu.async_copy(out_hbm.at[nidx_v.at[0]], nbuf0_v, sem0)

    def body(i, carry):
        j = 2 * i
        pltpu.make_async_copy(out_hbm.at[nidx_v.at[j]], nbuf0_v, sem0).wait()
        pltpu.async_copy(out_hbm.at[nidx_v.at[j + 1]], nbuf1_v, sem1)
        pltpu.async_copy(nbuf0_v, acc_sh.at[sidx_v.at[j]], ssem,
                         add=True).wait()
        pltpu.make_async_copy(out_hbm.at[nidx_v.at[j + 1]], nbuf1_v,
                              sem1).wait()

        @pl.when(i < NCH // 2 - 1)
        def _():
            pltpu.async_copy(out_hbm.at[nidx_v.at[j + 2]], nbuf0_v, sem0)

        pltpu.async_copy(nbuf1_v, acc_sh.at[sidx_v.at[j + 1]], ssem,
                         add=True).wait()
        return carry

    lax.fori_loop(0, NCH // 2, body, 0)
    plsc.subcore_barrier()
    pltpu.sync_copy(acc_sh.at[pl.ds(sid * BPW, BPW)], n_out.at[pl.ds(base, BPW)])


def _tc_reduce(t_ref, c_ref, n_ref, o_ref):
    t = t_ref[...]
    score = jnp.sum(t * c_ref[...], axis=1)
    neg = jnp.sum(t * n_ref[...], axis=1)
    loss = -(jnp.sum(jax.nn.log_sigmoid(score))
             + jnp.sum(jax.nn.log_sigmoid(-neg)))
    o_ref[...] = jnp.reshape(loss, (1, 1))


def kernel(in_embed, out_embed, target, context, neg_context):
    f32 = jnp.float32
    i32 = jnp.int32
    mesh = plsc.VectorSubcoreMesh(core_axis_name="c", subcore_axis_name="s",
                                  num_cores=NC)

    tgt = target.astype(i32)
    ctx = context.astype(i32)
    in3 = in_embed.reshape(VOCAB // 8, 8, EMB)
    out3 = out_embed.reshape(VOCAB // 8, 8, EMB)

    rows_fn = functools.partial(
        pl.kernel,
        mesh=mesh,
        compiler_params=pltpu.CompilerParams(use_tc_tiling_on_sc=True),
        out_type=[jax.ShapeDtypeStruct((B, EMB), f32)] * 2,
        scratch_types=[
            pltpu.VMEM((BPW,), i32),
            pltpu.VMEM((BPW,), i32),
            pltpu.SemaphoreType.DMA,
            pltpu.SemaphoreType.DMA,
        ],
    )(_sc_rows)
    t_rows, c_rows = rows_fn(in3, out3, tgt, ctx)

    neg2 = neg_context.astype(i32).reshape(B * NEG // GR, GR)
    local = jnp.repeat(jnp.arange(BPW, dtype=i32), NEG)
    scat2 = ((jnp.arange(NW, dtype=i32) // NC * BPW)[:, None]
             + local[None, :]).reshape(B * NEG // GR, GR)
    zeros = jnp.zeros((BPW, EMB), f32)

    neg_fn = functools.partial(
        pl.kernel,
        mesh=mesh,
        compiler_params=pltpu.CompilerParams(use_tc_tiling_on_sc=False),
        out_type=jax.ShapeDtypeStruct((B, EMB), f32),
        scratch_types=[
            pltpu.VMEM_SHARED((NS * BPW, EMB), f32),
            pltpu.VMEM((NCH, GR), i32),
            pltpu.VMEM((NCH, GR), i32),
            pltpu.VMEM((GR, EMB), f32),
            pltpu.VMEM((GR, EMB), f32),
            pltpu.SemaphoreType.DMA,
            pltpu.SemaphoreType.DMA,
            pltpu.SemaphoreType.DMA,
        ],
    )(_sc_neg)
    n_sum = neg_fn(out_embed, neg2, scat2, zeros)

    loss = pl.pallas_call(
        _tc_reduce,
        out_shape=jax.ShapeDtypeStruct((1, 1), f32),
    )(t_rows, c_rows, n_sum)
    return loss[0, 0]


# speedup vs baseline: 1.0786x; 1.0786x over previous
import functools

import jax
import jax.numpy as jnp
from jax import lax
from jax.experimental import pallas as pl
from jax.experimental.pallas import tpu as pltpu
from jax.experimental.pallas import tpu_sc as plsc

VOCAB = 1000000
EMB = 64
B = 16384
NEG = 20

NC = 2
NS = 16
NW = NC * NS
BPW = B // NW
GR = 128
NCH = BPW * NEG // GR


def _sc_rows(in_hbm, out_hbm, tgt_hbm, ctx_hbm, t_out, c_out,
             tidx_v, cidx_v, trows_v, crows_v, tsem, csem):
    wid = lax.axis_index("s") * NC + lax.axis_index("c")
    base = wid * BPW
    pltpu.sync_copy(tgt_hbm.at[pl.ds(wid * (BPW // GR), BPW // GR)], tidx_v)
    pltpu.sync_copy(ctx_hbm.at[pl.ds(wid * (BPW // GR), BPW // GR)], cidx_v)
    tcp = [pltpu.async_copy(in_hbm.at[tidx_v.at[j]],
                            trows_v.at[pl.ds(j * GR, GR)], tsem)
           for j in range(BPW // GR)]
    ccp = [pltpu.async_copy(out_hbm.at[cidx_v.at[j]],
                            crows_v.at[pl.ds(j * GR, GR)], csem)
           for j in range(BPW // GR)]
    for c in tcp:
        c.wait()
    pltpu.sync_copy(trows_v, t_out.at[pl.ds(base, BPW)])
    for c in ccp:
        c.wait()
    pltpu.sync_copy(crows_v, c_out.at[pl.ds(base, BPW)])


def _sc_neg(out_hbm, neg_hbm, scat_hbm, zer_hbm, n_out,
            acc_sh, nidx_v, sidx_v, nbuf0_v, nbuf1_v, sem0, sem1, ssem):
    sid = lax.axis_index("s")
    wid = sid * NC + lax.axis_index("c")
    base = wid * BPW

    pltpu.sync_copy(zer_hbm, acc_sh.at[pl.ds(sid * BPW, BPW)])
    pltpu.sync_copy(neg_hbm.at[pl.ds(wid * NCH, NCH)], nidx_v)
    pltpu.sync_copy(scat_hbm.at[pl.ds(wid * NCH, NCH)], sidx_v)
    plsc.subcore_barrier()

    pltpu.async_copy(out_hbm.at[nidx_v.at[0]], nbuf0_v, sem0)

    def body(i, carry):
        j = 2 * i
        pltpu.make_async_copy(out_hbm.at[nidx_v.at[j]], nbuf0_v, sem0).wait()
        pltpu.async_copy(out_hbm.at[nidx_v.at[j + 1]], nbuf1_v, sem1)
        pltpu.async_copy(nbuf0_v, acc_sh.at[sidx_v.at[j]], ssem,
                         add=True).wait()
        pltpu.make_async_copy(out_hbm.at[nidx_v.at[j + 1]], nbuf1_v,
                              sem1).wait()

        @pl.when(i < NCH // 2 - 1)
        def _():
            pltpu.async_copy(out_hbm.at[nidx_v.at[j + 2]], nbuf0_v, sem0)

        pltpu.async_copy(nbuf1_v, acc_sh.at[sidx_v.at[j + 1]], ssem,
                         add=True).wait()
        return carry

    lax.fori_loop(0, NCH // 2, body, 0)
    plsc.subcore_barrier()
    pltpu.sync_copy(acc_sh.at[pl.ds(sid * BPW, BPW)], n_out.at[pl.ds(base, BPW)])


def _tc_reduce(t_ref, c_ref, n_ref, o_ref):
    t = t_ref[...]
    score = jnp.sum(t * c_ref[...], axis=1)
    neg = jnp.sum(t * n_ref[...], axis=1)
    loss = -(jnp.sum(jax.nn.log_sigmoid(score))
             + jnp.sum(jax.nn.log_sigmoid(-neg)))
    o_ref[...] = jnp.reshape(loss, (1, 1))


def kernel(in_embed, out_embed, target, context, neg_context):
    f32 = jnp.float32
    i32 = jnp.int32
    mesh = plsc.VectorSubcoreMesh(core_axis_name="c", subcore_axis_name="s",
                                  num_cores=NC)

    tgt2 = target.astype(i32).reshape(B // GR, GR)
    ctx2 = context.astype(i32).reshape(B // GR, GR)

    rows_fn = functools.partial(
        pl.kernel,
        mesh=mesh,
        compiler_params=pltpu.CompilerParams(use_tc_tiling_on_sc=False),
        out_type=[jax.ShapeDtypeStruct((B, EMB), f32)] * 2,
        scratch_types=[
            pltpu.VMEM((BPW // GR, GR), i32),
            pltpu.VMEM((BPW // GR, GR), i32),
            pltpu.VMEM((BPW, EMB), f32),
            pltpu.VMEM((BPW, EMB), f32),
            pltpu.SemaphoreType.DMA,
            pltpu.SemaphoreType.DMA,
        ],
    )(_sc_rows)
    t_rows, c_rows = rows_fn(in_embed, out_embed, tgt2, ctx2)

    neg2 = neg_context.astype(i32).reshape(B * NEG // GR, GR)
    local = jnp.repeat(jnp.arange(BPW, dtype=i32), NEG)
    scat2 = ((jnp.arange(NW, dtype=i32) // NC * BPW)[:, None]
             + local[None, :]).reshape(B * NEG // GR, GR)
    zeros = jnp.zeros((BPW, EMB), f32)

    neg_fn = functools.partial(
        pl.kernel,
        mesh=mesh,
        compiler_params=pltpu.CompilerParams(use_tc_tiling_on_sc=False),
        out_type=jax.ShapeDtypeStruct((B, EMB), f32),
        scratch_types=[
            pltpu.VMEM_SHARED((NS * BPW, EMB), f32),
            pltpu.VMEM((NCH, GR), i32),
            pltpu.VMEM((NCH, GR), i32),
            pltpu.VMEM((GR, EMB), f32),
            pltpu.VMEM((GR, EMB), f32),
            pltpu.SemaphoreType.DMA,
            pltpu.SemaphoreType.DMA,
            pltpu.SemaphoreType.DMA,
        ],
    )(_sc_neg)
    n_sum = neg_fn(out_embed, neg2, scat2, zeros)

    loss = pl.pallas_call(
        _tc_reduce,
        out_shape=jax.ShapeDtypeStruct((1, 1), f32),
    )(t_rows, c_rows, n_sum)
    return loss[0, 0]
